# baseline (device time: 90206 ns/iter reference)
import jax
import jax.numpy as jnp
from jax import lax
from jax.experimental import pallas as pl
from jax.experimental.pallas import tpu as pltpu

N_DEV = 4


def kernel(x, dest):
    m, n = x.shape
    dr, dc = 8, 128
    dest2 = dest.reshape(dr, dc)

    def body(x_ref, d_ref, xg_ref, dg_ref,
             commx, commd, sendx, recvx, sendd, recvd):
        my_x = lax.axis_index("x")
        my_y = lax.axis_index("y")
        my_z = lax.axis_index("z")
        left = (my_z - 1) % N_DEV
        right = (my_z + 1) % N_DEV

        barrier_sem = pltpu.get_barrier_semaphore()
        for nbr in [left, right]:
            pl.semaphore_signal(
                barrier_sem, inc=1,
                device_id=(my_x, my_y, nbr),
                device_id_type=pl.DeviceIdType.MESH,
            )
        pl.semaphore_wait(barrier_sem, 2)

        xg_ref[pl.ds(my_z * m, m), :] = x_ref[...]
        dg_ref[pl.ds(my_z * dr, dr), :] = d_ref[...]
        commx[0] = x_ref[...]
        commd[0] = d_ref[...]

        for h in range(N_DEV - 1):
            s = h % 2
            r = (h + 1) % 2
            rx = pltpu.make_async_remote_copy(
                src_ref=commx.at[s], dst_ref=commx.at[r],
                send_sem=sendx.at[s], recv_sem=recvx.at[r],
                device_id=(my_x, my_y, right),
                device_id_type=pl.DeviceIdType.MESH,
            )
            rd = pltpu.make_async_remote_copy(
                src_ref=commd.at[s], dst_ref=commd.at[r],
                send_sem=sendd.at[s], recv_sem=recvd.at[r],
                device_id=(my_x, my_y, right),
                device_id_type=pl.DeviceIdType.MESH,
            )
            rx.start()
            rd.start()
            rx.wait()
            rd.wait()
            origin = (my_z - h - 1) % N_DEV
            xg_ref[pl.ds(origin * m, m), :] = commx[r]
            dg_ref[pl.ds(origin * dr, dr), :] = commd[r]

    xg, dg = pl.pallas_call(
        body,
        out_shape=[
            jax.ShapeDtypeStruct((N_DEV * m, n), jnp.float32),
            jax.ShapeDtypeStruct((N_DEV * dr, dc), jnp.int32),
        ],
        in_specs=[
            pl.BlockSpec(memory_space=pltpu.VMEM),
            pl.BlockSpec(memory_space=pltpu.VMEM),
        ],
        out_specs=[
            pl.BlockSpec(memory_space=pltpu.VMEM),
            pl.BlockSpec(memory_space=pltpu.VMEM),
        ],
        scratch_shapes=[
            pltpu.VMEM((2, m, n), jnp.float32),
            pltpu.VMEM((2, dr, dc), jnp.int32),
            pltpu.SemaphoreType.DMA((2,)),
            pltpu.SemaphoreType.DMA((2,)),
            pltpu.SemaphoreType.DMA((2,)),
            pltpu.SemaphoreType.DMA((2,)),
        ],
        compiler_params=pltpu.CompilerParams(collective_id=0),
    )(x, dest2)

    dfull = dg.reshape(-1)
    order = jnp.argsort(dfull, stable=True)
    my_z = lax.axis_index("z")
    my_idx = lax.dynamic_slice(order, (my_z * m,), (m,))
    return xg[my_idx]
